# baseline (speedup 1.0000x reference)
import jax
import jax.numpy as jnp
from jax.experimental import pallas as pl
from jax.experimental.pallas import tpu as pltpu

_B, _D, _RANK, _K = 16384, 64, 6, 32
_BM = 512


def _fused_kernel(x_ref, wsel_ref, bsel_ref, u_ref, v_ref, wg_ref, bg_ref,
                  out_ref):
    x = x_ref[...]
    xt = x.T
    zt = jax.lax.dot_general(wsel_ref[...], x, (((1,), (1,)), ((), ())),
                             preferred_element_type=jnp.float32)
    zt = zt + bsel_ref[...]

    row = jax.lax.broadcasted_iota(jnp.int32, (_D, _BM), 0)
    b = zt.view(jnp.int32)
    k = jnp.where(b >= 0, b, jnp.int32(-0x80000000) - b)

    kadj = jnp.where(row > 0, k - 1, k)
    rank = jnp.zeros((_D, _BM), jnp.int32)
    for i in range(_D):
        if i > 0:
            onehot = (row == i).astype(jnp.int32)
            kadj = kadj + onehot
        rank = rank + (kadj[i:i + 1, :] > kadj).astype(jnp.int32)
    xs = jnp.where(rank < _K, xt, 0.0)

    m = jnp.dot(u_ref[...], v_ref[...], preferred_element_type=jnp.float32)
    crosst = jax.lax.dot_general(m, xs, (((0,), (0,)), ((), ())),
                                 preferred_element_type=jnp.float32)
    scale = 1.0 / (_RANK ** 0.5)
    xi = xs * (1.0 + scale * crosst)

    g = jax.nn.sigmoid(
        jnp.sum(xi * wg_ref[...], axis=0, keepdims=True) + bg_ref[...])
    out_ref[...] = (g * xi + (1.0 - g) * xs).T


def kernel(x, W_sel, b_sel, U, V, W_gate, b_gate):
    b_sel2 = b_sel.reshape(_D, 1)
    wg2 = W_gate.reshape(_D, 1)
    b_gate2 = b_gate.reshape(1, 1)
    grid = (_B // _BM,)
    return pl.pallas_call(
        _fused_kernel,
        grid=grid,
        in_specs=[
            pl.BlockSpec((_BM, _D), lambda i: (i, 0)),
            pl.BlockSpec((_D, _D), lambda i: (0, 0)),
            pl.BlockSpec((_D, 1), lambda i: (0, 0)),
            pl.BlockSpec((_D, _RANK), lambda i: (0, 0)),
            pl.BlockSpec((_RANK, _D), lambda i: (0, 0)),
            pl.BlockSpec((_D, 1), lambda i: (0, 0)),
            pl.BlockSpec((1, 1), lambda i: (0, 0)),
        ],
        out_specs=pl.BlockSpec((_BM, _D), lambda i: (i, 0)),
        out_shape=jax.ShapeDtypeStruct((_B, _D), jnp.float32),
        compiler_params=pltpu.CompilerParams(
            dimension_semantics=("parallel",)),
    )(x, W_sel, b_sel2, U, V, wg2, b_gate2)


# speedup vs baseline: 1.0942x; 1.0942x over previous
import jax
import jax.numpy as jnp
from jax.experimental import pallas as pl
from jax.experimental.pallas import tpu as pltpu

_B, _D, _RANK, _K = 16384, 64, 6, 32
_BM = 1024


def _fused_kernel(x_ref, wsel_ref, bsel_ref, u_ref, v_ref, wg_ref, bg_ref,
                  out_ref):
    x = x_ref[...]
    xt = x.T
    zt = jax.lax.dot_general(wsel_ref[...], x, (((1,), (1,)), ((), ())),
                             preferred_element_type=jnp.float32)
    zt = zt + bsel_ref[...]

    row = jax.lax.broadcasted_iota(jnp.int32, (_D, _BM), 0)
    b = zt.view(jnp.int32)
    k = jnp.where(b >= 0, b, jnp.int32(-0x80000000) - b)

    kadj = jnp.where(row > 0, k - 1, k)
    rank = jnp.zeros((_D, _BM), jnp.int32)
    for i in range(_D):
        if i > 0:
            onehot = (row == i).astype(jnp.int32)
            kadj = kadj + onehot
        rank = rank + (kadj[i:i + 1, :] > kadj).astype(jnp.int32)
    xs = jnp.where(rank < _K, xt, 0.0)

    m = jnp.dot(u_ref[...], v_ref[...], preferred_element_type=jnp.float32)
    crosst = jax.lax.dot_general(m, xs, (((0,), (0,)), ((), ())),
                                 preferred_element_type=jnp.float32)
    scale = 1.0 / (_RANK ** 0.5)
    xi = xs * (1.0 + scale * crosst)

    g = jax.nn.sigmoid(
        jnp.sum(xi * wg_ref[...], axis=0, keepdims=True) + bg_ref[...])
    out_ref[...] = (g * xi + (1.0 - g) * xs).T


def kernel(x, W_sel, b_sel, U, V, W_gate, b_gate):
    b_sel2 = b_sel.reshape(_D, 1)
    wg2 = W_gate.reshape(_D, 1)
    b_gate2 = b_gate.reshape(1, 1)
    grid = (_B // _BM,)
    return pl.pallas_call(
        _fused_kernel,
        grid=grid,
        in_specs=[
            pl.BlockSpec((_BM, _D), lambda i: (i, 0)),
            pl.BlockSpec((_D, _D), lambda i: (0, 0)),
            pl.BlockSpec((_D, 1), lambda i: (0, 0)),
            pl.BlockSpec((_D, _RANK), lambda i: (0, 0)),
            pl.BlockSpec((_RANK, _D), lambda i: (0, 0)),
            pl.BlockSpec((_D, 1), lambda i: (0, 0)),
            pl.BlockSpec((1, 1), lambda i: (0, 0)),
        ],
        out_specs=pl.BlockSpec((_BM, _D), lambda i: (i, 0)),
        out_shape=jax.ShapeDtypeStruct((_B, _D), jnp.float32),
        compiler_params=pltpu.CompilerParams(
            dimension_semantics=("parallel",)),
    )(x, W_sel, b_sel2, U, V, wg2, b_gate2)
